# Initial kernel scaffold; baseline (speedup 1.0000x reference)
import jax
import jax.numpy as jnp
from jax.experimental import pallas as pl


def kernel(h, pos, edge_index, Wq, bq, Wk, bk, Wv, bv):
    raise NotImplementedError("write your pallas kernel here")


# speedup vs baseline: 2.1675x; 2.1675x over previous
import functools

import jax
import jax.numpy as jnp
from jax import lax
from jax.experimental import pallas as pl
from jax.experimental.pallas import tpu as pltpu
from jax.experimental.pallas import tpu_sc as plsc

N_NODES = 10000
N_EDGES = 320000
IN_DIM = 128
OUT_DIM = 16
NUM_HEADS = 8
HID = NUM_HEADS * OUT_DIM

NC = 2
NS = 16
NW = NC * NS
LANES = 16

E_PER_TILE = N_EDGES // NW
CA = 40
NA = E_PER_TILE // CA
CB = 40
NB = E_PER_TILE // CB
RNG = 640
N_PAD2 = 10240
N_RNG = N_PAD2 // RNG


def _proj_body(h_ref, wq_ref, wk_ref, wv_ref, bq_ref, bk_ref, bv_ref,
               cos_ref, sin_ref, q_out, k_out, v_out):
    x = h_ref[...]
    cosf = cos_ref[...]
    sinf = sin_ref[...]
    lane = lax.broadcasted_iota(jnp.int32, x.shape, 1)
    first_half = (lane % 8) < 4

    def rope(t):
        u = jnp.concatenate([t[:, 4:], t[:, :4]], axis=1)
        w = jnp.concatenate([t[:, -4:], t[:, :-4]], axis=1)
        rh = jnp.where(first_half, -u, w)
        return t * cosf + rh * sinf

    q = jnp.dot(x, wq_ref[...], preferred_element_type=jnp.float32) + bq_ref[...]
    k = jnp.dot(x, wk_ref[...], preferred_element_type=jnp.float32) + bk_ref[...]
    v = jnp.dot(x, wv_ref[...], preferred_element_type=jnp.float32) + bv_ref[...]
    q_out[...] = rope(q) * (1.0 / (OUT_DIM ** 0.5))
    k_out[...] = rope(k)
    v_out[...] = v


def _project(h2, wqt, wkt, wvt, bq, bk, bv, cosf, sinf, interpret=False):
    r = 2000
    bs_rows = pl.BlockSpec((r, IN_DIM), lambda i: (i, 0))
    bs_w = pl.BlockSpec((IN_DIM, HID), lambda i: (0, 0))
    bs_b = pl.BlockSpec((1, HID), lambda i: (0, 0))
    bs_out = pl.BlockSpec((r, HID), lambda i: (i, 0))
    return pl.pallas_call(
        _proj_body,
        grid=(N_NODES // r,),
        in_specs=[bs_rows, bs_w, bs_w, bs_w, bs_b, bs_b, bs_b, bs_out, bs_out],
        out_specs=[bs_out, bs_out, bs_out],
        out_shape=[jax.ShapeDtypeStruct((N_NODES, HID), jnp.float32)] * 3,
        interpret=interpret,
    )(h2, wqt, wkt, wvt, bq, bk, bv, cosf, sinf)


_GDN = lax.GatherDimensionNumbers(
    offset_dims=(), collapsed_slice_dims=(0,), start_index_map=(0,))


def _shuffle16(x, idx):
    return lax.gather(x, idx, _GDN, slice_sizes=(1,),
                      mode=lax.GatherScatterMode.PROMISE_IN_BOUNDS)


def _sum_all_lanes(x, perms):
    for idx in perms:
        x = x + _shuffle16(x, idx)
    return x


def _edge_body(k_hbm, q_hbm, v_hbm, src_hbm, dst_hbm,
               wv_e, z_e, wv_parts, z_parts,
               acc, zacc, src_idx, dst_idx,
               krows, qrows, vrows, zrows, wvb, zb, dstb, sem):
    cid = lax.axis_index("c")
    sid = lax.axis_index("s")
    wid = cid * NS + sid
    ebase = wid * E_PER_TILE
    lane = lax.iota(jnp.int32, LANES)
    zero16 = jnp.zeros((LANES,), jnp.float32)
    perms = [((lane ^ s).reshape(LANES, 1)) for s in (8, 4, 2, 1)]

    def chunk_a(t, c):
        base = ebase + t * CA
        pltpu.sync_copy(src_hbm.at[pl.ds(base, CA)], src_idx)
        pltpu.sync_copy(dst_hbm.at[pl.ds(base, CA)], dst_idx)
        d1 = pltpu.async_copy(k_hbm.at[src_idx], krows, sem)
        d2 = pltpu.async_copy(q_hbm.at[dst_idx], qrows, sem)
        d3 = pltpu.async_copy(v_hbm.at[src_idx], vrows, sem)
        d1.wait()
        d2.wait()
        d3.wait()

        def edge_a(i, c2):
            zvec = zero16
            for hh in range(NUM_HEADS):
                sl = pl.ds(hh * OUT_DIM, OUT_DIM)
                kq = krows[i, sl] * qrows[i, sl]
                sv = jnp.exp(jnp.clip(_sum_all_lanes(kq, perms), -5.0, 5.0))
                krows[i, sl] = sv * vrows[i, sl]
                zvec = jnp.where(lane == hh, sv, zvec)
            zrows[pl.ds(i * LANES, LANES)] = zvec
            return c2
        lax.fori_loop(0, CA, edge_a, 0)
        pltpu.sync_copy(krows, wv_e.at[pl.ds(base, CA)])
        pltpu.sync_copy(zrows, z_e.at[pl.ds(base * LANES, CA * LANES)])
        return c
    lax.fori_loop(0, NA, chunk_a, 0)

    def range_pass(r, c):
        lo = r * RNG

        def zero_acc(i, c2):
            for j in range(HID // LANES):
                acc[i, pl.ds(j * LANES, LANES)] = zero16
            zacc[pl.ds(i * LANES, LANES)] = zero16
            return c2
        lax.fori_loop(0, RNG, zero_acc, 0)

        def chunk_b(t, c2):
            base = ebase + t * CB
            pltpu.sync_copy(wv_e.at[pl.ds(base, CB)], wvb)
            pltpu.sync_copy(z_e.at[pl.ds(base * LANES, CB * LANES)], zb)
            pltpu.sync_copy(dst_hbm.at[pl.ds(base, CB)], dstb.at[pl.ds(0, CB)])

            def edge_b(i, c3):
                d = dstb[pl.ds(i, LANES)][0]
                dl = d - lo
                inr = (dl >= 0) & (dl < RNG)
                f = inr.astype(jnp.float32)
                dls = jnp.clip(dl, 0, RNG - 1)
                for hh in range(NUM_HEADS):
                    sl = pl.ds(hh * OUT_DIM, OUT_DIM)
                    acc[dls, sl] = acc[dls, sl] + f * wvb[i, sl]
                zsl = pl.ds(dls * LANES, LANES)
                zacc[zsl] = zacc[zsl] + f * zb[pl.ds(i * LANES, LANES)]
                return c3
            lax.fori_loop(0, CB, edge_b, 0)
            return c2
        lax.fori_loop(0, NB, chunk_b, 0)

        pltpu.sync_copy(acc, wv_parts.at[wid, pl.ds(lo, RNG)])
        pltpu.sync_copy(zacc, z_parts.at[wid, pl.ds(lo * LANES, RNG * LANES)])
        return c
    lax.fori_loop(0, N_RNG, range_pass, 0)


@functools.cache
def _make_edge_kernel():
    return functools.partial(
        pl.kernel,
        out_type=(
            jax.ShapeDtypeStruct((N_EDGES, HID), jnp.float32),
            jax.ShapeDtypeStruct((N_EDGES * LANES,), jnp.float32),
            jax.ShapeDtypeStruct((NW, N_PAD2, HID), jnp.float32),
            jax.ShapeDtypeStruct((NW, N_PAD2 * LANES), jnp.float32),
        ),
        mesh=plsc.VectorSubcoreMesh(
            core_axis_name="c", subcore_axis_name="s",
            num_cores=NC, num_subcores=NS,
        ),
        scratch_types=[
            pltpu.VMEM((RNG, HID), jnp.float32),
            pltpu.VMEM((RNG * LANES,), jnp.float32),
            pltpu.VMEM((CA,), jnp.int32),
            pltpu.VMEM((CA,), jnp.int32),
            pltpu.VMEM((CA, HID), jnp.float32),
            pltpu.VMEM((CA, HID), jnp.float32),
            pltpu.VMEM((CA, HID), jnp.float32),
            pltpu.VMEM((CA * LANES,), jnp.float32),
            pltpu.VMEM((CB, HID), jnp.float32),
            pltpu.VMEM((CB * LANES,), jnp.float32),
            pltpu.VMEM((CB + LANES,), jnp.int32),
            pltpu.SemaphoreType.DMA,
        ],
    )(_edge_body)


def _combine_body(wv_ref, z_ref, out_ref):
    wv = jnp.sum(wv_ref[...], axis=0)
    z = jnp.sum(z_ref[...], axis=0)
    head = lax.broadcasted_iota(jnp.int32, (LANES, HID), 1) // OUT_DIM
    row = lax.broadcasted_iota(jnp.int32, (LANES, HID), 0)
    sel = (head == row).astype(jnp.float32)
    den = jnp.dot(z, sel, preferred_element_type=jnp.float32)
    out_ref[...] = wv / den


def _combine(wv_parts, z_parts, interpret=False):
    r = 400
    bs_wv = pl.BlockSpec((NW, r, HID), lambda i: (0, i, 0))
    bs_z = pl.BlockSpec((NW, r, LANES), lambda i: (0, i, 0))
    bs_out = pl.BlockSpec((r, HID), lambda i: (i, 0))
    return pl.pallas_call(
        _combine_body,
        grid=(N_NODES // r,),
        in_specs=[bs_wv, bs_z],
        out_specs=bs_out,
        out_shape=jax.ShapeDtypeStruct((N_NODES, HID), jnp.float32),
        interpret=interpret,
    )(wv_parts, z_parts)


def _rotary_tables(pos):
    dim = OUT_DIM // 2
    inv_freq = 1.0 / (10000.0 ** (jnp.arange(0, dim, 2, dtype=jnp.float32) / dim))
    t = pos[0] * 64.0
    fx = t[:, 0:1] * inv_freq[None, :]
    fy = t[:, 1:2] * inv_freq[None, :]
    fx = jnp.concatenate([fx, fx], axis=1)
    fy = jnp.concatenate([fy, fy], axis=1)
    f = jnp.concatenate([fx, fy], axis=1)
    cosf = jnp.tile(jnp.cos(f), (1, NUM_HEADS))
    sinf = jnp.tile(jnp.sin(f), (1, NUM_HEADS))
    return cosf, sinf


def kernel(h, pos, edge_index, Wq, bq, Wk, bk, Wv, bv):
    h2 = h[0]
    cosf, sinf = _rotary_tables(pos)
    qr, kr, vr = _project(
        h2, Wq.T, Wk.T, Wv.T,
        bq.reshape(1, HID), bk.reshape(1, HID), bv.reshape(1, HID),
        cosf, sinf,
    )
    src = edge_index[0].astype(jnp.int32)
    dst = edge_index[1].astype(jnp.int32)
    _, _, wv_parts, z_parts = _make_edge_kernel()(kr, qr, vr, src, dst)
    z3 = z_parts.reshape(NW, N_PAD2, LANES)
    out = _combine(wv_parts[:, :N_NODES], z3[:, :N_NODES])
    return out[None]


# speedup vs baseline: 2.2112x; 1.0202x over previous
import functools

import jax
import jax.numpy as jnp
from jax import lax
from jax.experimental import pallas as pl
from jax.experimental.pallas import tpu as pltpu
from jax.experimental.pallas import tpu_sc as plsc

N_NODES = 10000
N_EDGES = 320000
IN_DIM = 128
OUT_DIM = 16
NUM_HEADS = 8
HID = NUM_HEADS * OUT_DIM

NC = 2
NS = 16
NW = NC * NS
LANES = 16

E_PER_TILE = N_EDGES // NW
CA = 40
NA = E_PER_TILE // CA
CB = 80
NB = E_PER_TILE // CB
RNG = 512
N_PAD2 = 10240
N_RNG = N_PAD2 // RNG


def _proj_body(h_ref, wq_ref, wk_ref, wv_ref, bq_ref, bk_ref, bv_ref,
               cos_ref, sin_ref, q_out, k_out, v_out):
    x = h_ref[...]
    cosf = cos_ref[...]
    sinf = sin_ref[...]
    lane = lax.broadcasted_iota(jnp.int32, x.shape, 1)
    first_half = (lane % 8) < 4

    def rope(t):
        u = jnp.concatenate([t[:, 4:], t[:, :4]], axis=1)
        w = jnp.concatenate([t[:, -4:], t[:, :-4]], axis=1)
        rh = jnp.where(first_half, -u, w)
        return t * cosf + rh * sinf

    q = jnp.dot(x, wq_ref[...], preferred_element_type=jnp.float32) + bq_ref[...]
    k = jnp.dot(x, wk_ref[...], preferred_element_type=jnp.float32) + bk_ref[...]
    v = jnp.dot(x, wv_ref[...], preferred_element_type=jnp.float32) + bv_ref[...]
    q_out[...] = rope(q) * (1.0 / (OUT_DIM ** 0.5))
    k_out[...] = rope(k)
    v_out[...] = v


def _project(h2, wqt, wkt, wvt, bq, bk, bv, cosf, sinf, interpret=False):
    r = 2000
    bs_rows = pl.BlockSpec((r, IN_DIM), lambda i: (i, 0))
    bs_w = pl.BlockSpec((IN_DIM, HID), lambda i: (0, 0))
    bs_b = pl.BlockSpec((1, HID), lambda i: (0, 0))
    bs_out = pl.BlockSpec((r, HID), lambda i: (i, 0))
    return pl.pallas_call(
        _proj_body,
        grid=(N_NODES // r,),
        in_specs=[bs_rows, bs_w, bs_w, bs_w, bs_b, bs_b, bs_b, bs_out, bs_out],
        out_specs=[bs_out, bs_out, bs_out],
        out_shape=[jax.ShapeDtypeStruct((N_NODES, HID), jnp.float32)] * 3,
        interpret=interpret,
    )(h2, wqt, wkt, wvt, bq, bk, bv, cosf, sinf)


_GDN = lax.GatherDimensionNumbers(
    offset_dims=(), collapsed_slice_dims=(0,), start_index_map=(0,))


def _shuffle16(x, idx):
    return lax.gather(x, idx, _GDN, slice_sizes=(1,),
                      mode=lax.GatherScatterMode.PROMISE_IN_BOUNDS)


def _sum_all_lanes(x, perms):
    for idx in perms:
        x = x + _shuffle16(x, idx)
    return x


def _edge_body(k_hbm, q_hbm, v_hbm, src_hbm, dst_hbm,
               wv_e, z_e, wv_parts, z_parts,
               acc, zacc, src_idx, dst_all,
               krows, qrows, vrows, zrows, wvb, zb, sem):
    cid = lax.axis_index("c")
    sid = lax.axis_index("s")
    wid = cid * NS + sid
    ebase = wid * E_PER_TILE
    lane = lax.iota(jnp.int32, LANES)
    zero16 = jnp.zeros((LANES,), jnp.float32)
    perms = [((lane ^ s).reshape(LANES, 1)) for s in (8, 4, 2, 1)]

    pltpu.sync_copy(dst_hbm.at[pl.ds(ebase, E_PER_TILE)],
                    dst_all.at[pl.ds(0, E_PER_TILE)])

    def chunk_a(t, c):
        base = ebase + t * CA
        pltpu.sync_copy(src_hbm.at[pl.ds(base, CA)], src_idx)
        d1 = pltpu.async_copy(k_hbm.at[src_idx], krows, sem)
        d2 = pltpu.async_copy(
            q_hbm.at[dst_all.at[pl.ds(t * CA, CA)]], qrows, sem)
        d3 = pltpu.async_copy(v_hbm.at[src_idx], vrows, sem)
        d1.wait()
        d2.wait()
        d3.wait()

        def edge_a(i, c2):
            zvec = zero16
            for hh in range(NUM_HEADS):
                sl = pl.ds(hh * OUT_DIM, OUT_DIM)
                kq = krows[i, sl] * qrows[i, sl]
                sv = jnp.exp(jnp.clip(_sum_all_lanes(kq, perms), -5.0, 5.0))
                krows[i, sl] = sv * vrows[i, sl]
                zvec = jnp.where(lane == hh, sv, zvec)
            zrows[pl.ds(i * LANES, LANES)] = zvec
            return c2
        lax.fori_loop(0, CA, edge_a, 0)
        pltpu.sync_copy(krows, wv_e.at[pl.ds(base, CA)])
        pltpu.sync_copy(zrows, z_e.at[pl.ds(base * LANES, CA * LANES)])
        return c
    lax.fori_loop(0, NA, chunk_a, 0)

    def range_pass(r, c):
        lo = r * RNG

        def zero_acc(i, c2):
            for j in range(HID // LANES):
                acc[i, pl.ds(j * LANES, LANES)] = zero16
            zacc[pl.ds(i * LANES, LANES)] = zero16
            return c2
        lax.fori_loop(0, RNG, zero_acc, 0)

        def chunk_b(t, c2):
            base = ebase + t * CB
            pltpu.sync_copy(wv_e.at[pl.ds(base, CB)], wvb)
            pltpu.sync_copy(z_e.at[pl.ds(base * LANES, CB * LANES)], zb)

            def edge_b(i, c3):
                d = dst_all[pl.ds(t * CB + i, LANES)][0]
                dl = d - lo
                inr = (dl >= 0) & (dl < RNG)
                f = inr.astype(jnp.float32)
                dls = jnp.clip(dl, 0, RNG - 1)
                for hh in range(NUM_HEADS):
                    sl = pl.ds(hh * OUT_DIM, OUT_DIM)
                    acc[dls, sl] = acc[dls, sl] + f * wvb[i, sl]
                zsl = pl.ds(dls * LANES, LANES)
                zacc[zsl] = zacc[zsl] + f * zb[pl.ds(i * LANES, LANES)]
                return c3
            lax.fori_loop(0, CB, edge_b, 0)
            return c2
        lax.fori_loop(0, NB, chunk_b, 0)

        pltpu.sync_copy(acc, wv_parts.at[wid, pl.ds(lo, RNG)])
        pltpu.sync_copy(zacc, z_parts.at[wid, pl.ds(lo * LANES, RNG * LANES)])
        return c
    lax.fori_loop(0, N_RNG, range_pass, 0)


@functools.cache
def _make_edge_kernel():
    return functools.partial(
        pl.kernel,
        out_type=(
            jax.ShapeDtypeStruct((N_EDGES, HID), jnp.float32),
            jax.ShapeDtypeStruct((N_EDGES * LANES,), jnp.float32),
            jax.ShapeDtypeStruct((NW, N_PAD2, HID), jnp.float32),
            jax.ShapeDtypeStruct((NW, N_PAD2 * LANES), jnp.float32),
        ),
        mesh=plsc.VectorSubcoreMesh(
            core_axis_name="c", subcore_axis_name="s",
            num_cores=NC, num_subcores=NS,
        ),
        scratch_types=[
            pltpu.VMEM((RNG, HID), jnp.float32),
            pltpu.VMEM((RNG * LANES,), jnp.float32),
            pltpu.VMEM((CA,), jnp.int32),
            pltpu.VMEM((E_PER_TILE + LANES,), jnp.int32),
            pltpu.VMEM((CA, HID), jnp.float32),
            pltpu.VMEM((CA, HID), jnp.float32),
            pltpu.VMEM((CA, HID), jnp.float32),
            pltpu.VMEM((CA * LANES,), jnp.float32),
            pltpu.VMEM((CB, HID), jnp.float32),
            pltpu.VMEM((CB * LANES,), jnp.float32),
            pltpu.SemaphoreType.DMA,
        ],
    )(_edge_body)


def _combine_body(wv_ref, z_ref, out_ref):
    wv = jnp.sum(wv_ref[...], axis=0)
    z = jnp.sum(z_ref[...], axis=0)
    head = lax.broadcasted_iota(jnp.int32, (LANES, HID), 1) // OUT_DIM
    row = lax.broadcasted_iota(jnp.int32, (LANES, HID), 0)
    sel = (head == row).astype(jnp.float32)
    den = jnp.dot(z, sel, preferred_element_type=jnp.float32)
    out_ref[...] = wv / den


def _combine(wv_parts, z_parts, interpret=False):
    r = 400
    bs_wv = pl.BlockSpec((NW, r, HID), lambda i: (0, i, 0))
    bs_z = pl.BlockSpec((NW, r, LANES), lambda i: (0, i, 0))
    bs_out = pl.BlockSpec((r, HID), lambda i: (i, 0))
    return pl.pallas_call(
        _combine_body,
        grid=(N_NODES // r,),
        in_specs=[bs_wv, bs_z],
        out_specs=bs_out,
        out_shape=jax.ShapeDtypeStruct((N_NODES, HID), jnp.float32),
        interpret=interpret,
    )(wv_parts, z_parts)


def _rotary_tables(pos):
    dim = OUT_DIM // 2
    inv_freq = 1.0 / (10000.0 ** (jnp.arange(0, dim, 2, dtype=jnp.float32) / dim))
    t = pos[0] * 64.0
    fx = t[:, 0:1] * inv_freq[None, :]
    fy = t[:, 1:2] * inv_freq[None, :]
    fx = jnp.concatenate([fx, fx], axis=1)
    fy = jnp.concatenate([fy, fy], axis=1)
    f = jnp.concatenate([fx, fy], axis=1)
    cosf = jnp.tile(jnp.cos(f), (1, NUM_HEADS))
    sinf = jnp.tile(jnp.sin(f), (1, NUM_HEADS))
    return cosf, sinf


def kernel(h, pos, edge_index, Wq, bq, Wk, bk, Wv, bv):
    h2 = h[0]
    cosf, sinf = _rotary_tables(pos)
    qr, kr, vr = _project(
        h2, Wq.T, Wk.T, Wv.T,
        bq.reshape(1, HID), bk.reshape(1, HID), bv.reshape(1, HID),
        cosf, sinf,
    )
    src = edge_index[0].astype(jnp.int32)
    dst = edge_index[1].astype(jnp.int32)
    _, _, wv_parts, z_parts = _make_edge_kernel()(kr, qr, vr, src, dst)
    z3 = z_parts.reshape(NW, N_PAD2, LANES)
    out = _combine(wv_parts[:, :N_NODES], z3[:, :N_NODES])
    return out[None]


# speedup vs baseline: 2.9334x; 1.3266x over previous
import functools

import jax
import jax.numpy as jnp
from jax import lax
from jax.experimental import pallas as pl
from jax.experimental.pallas import tpu as pltpu
from jax.experimental.pallas import tpu_sc as plsc

N_NODES = 10000
N_EDGES = 320000
IN_DIM = 128
OUT_DIM = 16
NUM_HEADS = 8
HID = NUM_HEADS * OUT_DIM

NC = 2
NS = 16
NW = NC * NS
LANES = 16

E_PER_TILE = N_EDGES // NW
CA = 40
NA = E_PER_TILE // CA
CB = 80
NB = E_PER_TILE // CB
RNG = 512
N_PAD2 = 10240
N_RNG = N_PAD2 // RNG


def _proj_body(h_ref, wq_ref, wk_ref, wv_ref, bq_ref, bk_ref, bv_ref,
               cos_ref, sin_ref, q_out, k_out, v_out):
    x = h_ref[...]
    cosf = cos_ref[...]
    sinf = sin_ref[...]
    lane = lax.broadcasted_iota(jnp.int32, x.shape, 1)
    first_half = (lane % 8) < 4

    def rope(t):
        u = jnp.concatenate([t[:, 4:], t[:, :4]], axis=1)
        w = jnp.concatenate([t[:, -4:], t[:, :-4]], axis=1)
        rh = jnp.where(first_half, -u, w)
        return t * cosf + rh * sinf

    q = jnp.dot(x, wq_ref[...], preferred_element_type=jnp.float32) + bq_ref[...]
    k = jnp.dot(x, wk_ref[...], preferred_element_type=jnp.float32) + bk_ref[...]
    v = jnp.dot(x, wv_ref[...], preferred_element_type=jnp.float32) + bv_ref[...]
    q_out[...] = rope(q) * (1.0 / (OUT_DIM ** 0.5))
    k_out[...] = rope(k)
    v_out[...] = v


def _project(h2, wqt, wkt, wvt, bq, bk, bv, cosf, sinf, interpret=False):
    r = 2000
    bs_rows = pl.BlockSpec((r, IN_DIM), lambda i: (i, 0))
    bs_w = pl.BlockSpec((IN_DIM, HID), lambda i: (0, 0))
    bs_b = pl.BlockSpec((1, HID), lambda i: (0, 0))
    bs_out = pl.BlockSpec((r, HID), lambda i: (i, 0))
    return pl.pallas_call(
        _proj_body,
        grid=(N_NODES // r,),
        in_specs=[bs_rows, bs_w, bs_w, bs_w, bs_b, bs_b, bs_b, bs_out, bs_out],
        out_specs=[bs_out, bs_out, bs_out],
        out_shape=[jax.ShapeDtypeStruct((N_NODES, HID), jnp.float32)] * 3,
        interpret=interpret,
    )(h2, wqt, wkt, wvt, bq, bk, bv, cosf, sinf)


_GDN = lax.GatherDimensionNumbers(
    offset_dims=(), collapsed_slice_dims=(0,), start_index_map=(0,))


def _shuffle16(x, idx):
    return lax.gather(x, idx, _GDN, slice_sizes=(1,),
                      mode=lax.GatherScatterMode.PROMISE_IN_BOUNDS)


def _sum_all_lanes(x, perms):
    for idx in perms:
        x = x + _shuffle16(x, idx)
    return x


def _edge_body(k_hbm, q_hbm, v_hbm, src_hbm, dst_hbm,
               wv_e, z_e, wv_parts, z_parts,
               acc, zacc, src_idx, dst_all,
               krows, qrows, vrows, zrows, wvb, zb, sem):
    cid = lax.axis_index("c")
    sid = lax.axis_index("s")
    wid = cid * NS + sid
    ebase = wid * E_PER_TILE
    lane = lax.iota(jnp.int32, LANES)
    zero16 = jnp.zeros((LANES,), jnp.float32)
    perms = [((lane ^ s).reshape(LANES, 1)) for s in (8, 4, 2, 1)]

    pltpu.sync_copy(dst_hbm.at[pl.ds(ebase, E_PER_TILE)],
                    dst_all.at[pl.ds(0, E_PER_TILE)])

    def chunk_a(t, c):
        base = ebase + t * CA
        pltpu.sync_copy(src_hbm.at[pl.ds(base, CA)], src_idx)
        d1 = pltpu.async_copy(k_hbm.at[src_idx], krows, sem)
        d2 = pltpu.async_copy(
            q_hbm.at[dst_all.at[pl.ds(t * CA, CA)]], qrows, sem)
        d3 = pltpu.async_copy(v_hbm.at[src_idx], vrows, sem)
        d1.wait()
        d2.wait()
        d3.wait()

        def edge_a(i, c2):
            zvec = zero16
            for hh in range(NUM_HEADS):
                sl = pl.ds(hh * OUT_DIM, OUT_DIM)
                kq = krows[i, sl] * qrows[i, sl]
                sv = jnp.exp(jnp.clip(_sum_all_lanes(kq, perms), -5.0, 5.0))
                krows[i, sl] = sv * vrows[i, sl]
                zvec = jnp.where(lane == hh, sv, zvec)
            zrows[pl.ds(i * LANES, LANES)] = zvec
            return c2
        lax.fori_loop(0, CA, edge_a, 0)
        pltpu.sync_copy(krows, wv_e.at[pl.ds(base, CA)])
        pltpu.sync_copy(zrows, z_e.at[pl.ds(base * LANES, CA * LANES)])
        return c
    lax.fori_loop(0, NA, chunk_a, 0)

    def range_pass(r, c):
        lo = r * RNG

        def zero_acc(i, c2):
            for j in range(HID // LANES):
                acc[i, pl.ds(j * LANES, LANES)] = zero16
            zacc[pl.ds(i * LANES, LANES)] = zero16
            return c2
        lax.fori_loop(0, RNG + 1, zero_acc, 0)

        def chunk_b(t, c2):
            base = ebase + t * CB
            pltpu.sync_copy(wv_e.at[pl.ds(base, CB)], wvb)
            pltpu.sync_copy(z_e.at[pl.ds(base * LANES, CB * LANES)], zb)

            def group_b(g, c3):
                dvec = dst_all[pl.ds(t * CB + g * LANES, LANES)] - lo
                for i2 in range(LANES):
                    i = g * LANES + i2
                    dl = dvec[i2]
                    inr = (dl >= 0) & (dl < RNG)
                    dls = jnp.where(inr, jnp.clip(dl, 0, RNG - 1),
                                    jnp.int32(RNG))
                    for hh in range(NUM_HEADS):
                        sl = pl.ds(hh * OUT_DIM, OUT_DIM)
                        acc[dls, sl] = acc[dls, sl] + wvb[i, sl]
                    zsl = pl.ds(dls * LANES, LANES)
                    zacc[zsl] = zacc[zsl] + zb[pl.ds(i * LANES, LANES)]
                return c3
            lax.fori_loop(0, CB // LANES, group_b, 0)
            return c2
        lax.fori_loop(0, NB, chunk_b, 0)

        pltpu.sync_copy(acc.at[pl.ds(0, RNG)],
                        wv_parts.at[wid, pl.ds(lo, RNG)])
        pltpu.sync_copy(zacc.at[pl.ds(0, RNG * LANES)],
                        z_parts.at[wid, pl.ds(lo * LANES, RNG * LANES)])
        return c
    lax.fori_loop(0, N_RNG, range_pass, 0)


@functools.cache
def _make_edge_kernel():
    return functools.partial(
        pl.kernel,
        out_type=(
            jax.ShapeDtypeStruct((N_EDGES, HID), jnp.float32),
            jax.ShapeDtypeStruct((N_EDGES * LANES,), jnp.float32),
            jax.ShapeDtypeStruct((NW, N_PAD2, HID), jnp.float32),
            jax.ShapeDtypeStruct((NW, N_PAD2 * LANES), jnp.float32),
        ),
        mesh=plsc.VectorSubcoreMesh(
            core_axis_name="c", subcore_axis_name="s",
            num_cores=NC, num_subcores=NS,
        ),
        scratch_types=[
            pltpu.VMEM((RNG + 8, HID), jnp.float32),
            pltpu.VMEM(((RNG + 8) * LANES,), jnp.float32),
            pltpu.VMEM((CA,), jnp.int32),
            pltpu.VMEM((E_PER_TILE + LANES,), jnp.int32),
            pltpu.VMEM((CA, HID), jnp.float32),
            pltpu.VMEM((CA, HID), jnp.float32),
            pltpu.VMEM((CA, HID), jnp.float32),
            pltpu.VMEM((CA * LANES,), jnp.float32),
            pltpu.VMEM((CB, HID), jnp.float32),
            pltpu.VMEM((CB * LANES,), jnp.float32),
            pltpu.SemaphoreType.DMA,
        ],
    )(_edge_body)


def _combine_body(wv_ref, z_ref, out_ref):
    wv = jnp.sum(wv_ref[...], axis=0)
    z = jnp.sum(z_ref[...], axis=0)
    head = lax.broadcasted_iota(jnp.int32, (LANES, HID), 1) // OUT_DIM
    row = lax.broadcasted_iota(jnp.int32, (LANES, HID), 0)
    sel = (head == row).astype(jnp.float32)
    den = jnp.dot(z, sel, preferred_element_type=jnp.float32)
    out_ref[...] = wv / den


def _combine(wv_parts, z_parts, interpret=False):
    r = 400
    bs_wv = pl.BlockSpec((NW, r, HID), lambda i: (0, i, 0))
    bs_z = pl.BlockSpec((NW, r, LANES), lambda i: (0, i, 0))
    bs_out = pl.BlockSpec((r, HID), lambda i: (i, 0))
    return pl.pallas_call(
        _combine_body,
        grid=(N_NODES // r,),
        in_specs=[bs_wv, bs_z],
        out_specs=bs_out,
        out_shape=jax.ShapeDtypeStruct((N_NODES, HID), jnp.float32),
        interpret=interpret,
    )(wv_parts, z_parts)


def _rotary_tables(pos):
    dim = OUT_DIM // 2
    inv_freq = 1.0 / (10000.0 ** (jnp.arange(0, dim, 2, dtype=jnp.float32) / dim))
    t = pos[0] * 64.0
    fx = t[:, 0:1] * inv_freq[None, :]
    fy = t[:, 1:2] * inv_freq[None, :]
    fx = jnp.concatenate([fx, fx], axis=1)
    fy = jnp.concatenate([fy, fy], axis=1)
    f = jnp.concatenate([fx, fy], axis=1)
    cosf = jnp.tile(jnp.cos(f), (1, NUM_HEADS))
    sinf = jnp.tile(jnp.sin(f), (1, NUM_HEADS))
    return cosf, sinf


def kernel(h, pos, edge_index, Wq, bq, Wk, bk, Wv, bv):
    h2 = h[0]
    cosf, sinf = _rotary_tables(pos)
    qr, kr, vr = _project(
        h2, Wq.T, Wk.T, Wv.T,
        bq.reshape(1, HID), bk.reshape(1, HID), bv.reshape(1, HID),
        cosf, sinf,
    )
    src = edge_index[0].astype(jnp.int32)
    dst = edge_index[1].astype(jnp.int32)
    _, _, wv_parts, z_parts = _make_edge_kernel()(kr, qr, vr, src, dst)
    z3 = z_parts.reshape(NW, N_PAD2, LANES)
    out = _combine(wv_parts[:, :N_NODES], z3[:, :N_NODES])
    return out[None]
